# baseline (speedup 1.0000x reference)
import jax
import jax.numpy as jnp
from jax.experimental import pallas as pl
from jax.experimental.pallas import tpu as pltpu

HIDDEN = 4096
NUM_EXPERTS = 64
TOP_K = 8
TOKENS = 16384
BLOCK = 1024
NCHUNK = 8
RCHUNK = BLOCK // NCHUNK


def _router_body(*refs):
    x_refs = refs[:NCHUNK]
    wt_ref = refs[NCHUNK]
    scores_ref, w_ref, idx_ref = refs[NCHUNK + 1:]

    logits = jnp.concatenate([
        jax.lax.dot_general(
            xr[...], wt_ref[...],
            dimension_numbers=(((1,), (0,)), ((), ())),
            preferred_element_type=jnp.float32,
            precision=jax.lax.Precision.DEFAULT,
        )
        for xr in x_refs
    ], axis=0)
    lt = logits.T
    m = jnp.max(lt, axis=0, keepdims=True)
    e = jnp.exp(lt - m)
    s = jnp.sum(e, axis=0, keepdims=True)
    scores_t = e / s
    scores_ref[...] = scores_t.T

    iota = jax.lax.broadcasted_iota(jnp.int32, scores_t.shape, 0)
    cur = scores_t
    vals = []
    idxs = []
    for _ in range(TOP_K):
        mx = jnp.max(cur, axis=0, keepdims=True)
        amx = jnp.min(jnp.where(cur == mx, iota, NUM_EXPERTS),
                      axis=0, keepdims=True)
        vals.append(mx)
        idxs.append(amx)
        cur = jnp.where(iota == amx, -1.0, cur)
    v = jnp.concatenate(vals, axis=0)
    ii = jnp.concatenate(idxs, axis=0)
    norm = jnp.sum(v, axis=0, keepdims=True)
    w_ref[...] = (v / norm).T
    idx_ref[...] = ii.T


def _x_spec(j):
    return pl.BlockSpec((RCHUNK, HIDDEN), lambda i, j=j: (i * NCHUNK + j, 0))


def kernel(x, W):
    wt = W.T
    grid = (TOKENS // BLOCK,)
    scores, weights, top_experts = pl.pallas_call(
        _router_body,
        grid=grid,
        in_specs=[_x_spec(j) for j in range(NCHUNK)] + [
            pl.BlockSpec((HIDDEN, NUM_EXPERTS), lambda i: (0, 0)),
        ],
        out_specs=[
            pl.BlockSpec((BLOCK, NUM_EXPERTS), lambda i: (i, 0)),
            pl.BlockSpec((BLOCK, TOP_K), lambda i: (i, 0)),
            pl.BlockSpec((BLOCK, TOP_K), lambda i: (i, 0)),
        ],
        out_shape=[
            jax.ShapeDtypeStruct((TOKENS, NUM_EXPERTS), jnp.float32),
            jax.ShapeDtypeStruct((TOKENS, TOP_K), jnp.float32),
            jax.ShapeDtypeStruct((TOKENS, TOP_K), jnp.int32),
        ],
        compiler_params=pltpu.CompilerParams(
            dimension_semantics=("arbitrary",),
        ),
    )(*([x] * NCHUNK), wt)
    return (scores, weights, top_experts)


# speedup vs baseline: 1.0001x; 1.0001x over previous
import jax
import jax.numpy as jnp
from jax.experimental import pallas as pl
from jax.experimental.pallas import tpu as pltpu

HIDDEN = 4096
NUM_EXPERTS = 64
TOP_K = 8
TOKENS = 16384
BLOCK = 1024


def _router_body(x_ref, wt_ref, scores_ref, w_ref, idx_ref):
    logits = jax.lax.dot_general(
        x_ref[...], wt_ref[...],
        dimension_numbers=(((1,), (0,)), ((), ())),
        preferred_element_type=jnp.float32,
        precision=jax.lax.Precision.DEFAULT,
    )
    lt = logits.T
    m = jnp.max(lt, axis=0, keepdims=True)
    e = jnp.exp(lt - m)
    s = jnp.sum(e, axis=0, keepdims=True)
    scores_t = e / s
    scores_ref[...] = scores_t.T

    iota = jax.lax.broadcasted_iota(jnp.int32, scores_t.shape, 0)
    cur = scores_t
    vals = []
    idxs = []
    for _ in range(TOP_K):
        mx = jnp.max(cur, axis=0, keepdims=True)
        amx = jnp.min(jnp.where(cur == mx, iota, NUM_EXPERTS),
                      axis=0, keepdims=True)
        vals.append(mx)
        idxs.append(amx)
        cur = jnp.where(iota == amx, -1.0, cur)
    v = jnp.concatenate(vals, axis=0)
    ii = jnp.concatenate(idxs, axis=0)
    norm = jnp.sum(v, axis=0, keepdims=True)
    w_ref[...] = (v / norm).T
    idx_ref[...] = ii.T


def kernel(x, W):
    wt = W.T
    grid = (TOKENS // BLOCK,)
    scores, weights, top_experts = pl.pallas_call(
        _router_body,
        grid=grid,
        in_specs=[
            pl.BlockSpec((BLOCK, HIDDEN), lambda i: (i, 0)),
            pl.BlockSpec((HIDDEN, NUM_EXPERTS), lambda i: (0, 0)),
        ],
        out_specs=[
            pl.BlockSpec((BLOCK, NUM_EXPERTS), lambda i: (i, 0)),
            pl.BlockSpec((BLOCK, TOP_K), lambda i: (i, 0)),
            pl.BlockSpec((BLOCK, TOP_K), lambda i: (i, 0)),
        ],
        out_shape=[
            jax.ShapeDtypeStruct((TOKENS, NUM_EXPERTS), jnp.float32),
            jax.ShapeDtypeStruct((TOKENS, TOP_K), jnp.float32),
            jax.ShapeDtypeStruct((TOKENS, TOP_K), jnp.int32),
        ],
        compiler_params=pltpu.CompilerParams(
            dimension_semantics=("arbitrary",),
        ),
    )(x, wt)
    return (scores, weights, top_experts)
